# baseline (speedup 1.0000x reference)
import functools

import jax
import jax.numpy as jnp
from jax import lax
from jax.experimental import pallas as pl
from jax.experimental.pallas import tpu as pltpu
from jax.experimental.pallas import tpu_sc as plsc

MODEL_DIM = 64
SEQ_LEN = 200
BATCH = 4096
TOTAL = BATCH * SEQ_LEN
LANES = 16
D_VECS = MODEL_DIM // LANES

_info = plsc.get_sparse_core_info()
NC, NS = _info.num_cores, _info.num_subcores
NW = NC * NS
PER_W = TOTAL // NW
CHUNK = 512
GATHER_SPLIT = 128
NCHUNK = PER_W // CHUNK
SCALE = float(MODEL_DIM) ** 0.5

_mesh = plsc.VectorSubcoreMesh(core_axis_name="c", subcore_axis_name="s")


@functools.partial(
    pl.kernel,
    out_type=jax.ShapeDtypeStruct((TOTAL, MODEL_DIM), jnp.float32),
    mesh=_mesh,
    compiler_params=pltpu.CompilerParams(use_tc_tiling_on_sc=False),
    scratch_types=[
        pltpu.VMEM((CHUNK,), jnp.int32),
        pltpu.VMEM((CHUNK, MODEL_DIM), jnp.float32),
        pltpu.VMEM((SEQ_LEN, MODEL_DIM), jnp.float32),
        pltpu.SemaphoreType.DMA,
    ],
)
def _encode(x_hbm, table_hbm, pos_hbm, out_hbm, idx_v, rows_v, pos_v, sem):
    wid = lax.axis_index("s") * NC + lax.axis_index("c")
    pltpu.sync_copy(pos_hbm, pos_v)

    def chunk_body(c, carry):
        base = wid * PER_W + c * CHUNK
        pltpu.sync_copy(x_hbm.at[pl.ds(base, CHUNK)], idx_v)
        copies = [
            pltpu.async_copy(
                table_hbm.at[idx_v.at[pl.ds(g * GATHER_SPLIT, GATHER_SPLIT)]],
                rows_v.at[pl.ds(g * GATHER_SPLIT, GATHER_SPLIT)],
                sem,
            )
            for g in range(CHUNK // GATHER_SPLIT)
        ]
        for cp in copies:
            cp.wait()

        def row_body(j, carry2):
            p = lax.rem(base + j, SEQ_LEN)
            for d in range(D_VECS):
                pv = pos_v[p, pl.ds(d * LANES, LANES)]
                v = rows_v[j, pl.ds(d * LANES, LANES)]
                rows_v[j, pl.ds(d * LANES, LANES)] = v * SCALE + pv
            return carry2

        lax.fori_loop(0, CHUNK, row_body, 0)
        pltpu.sync_copy(rows_v, out_hbm.at[pl.ds(base, CHUNK)])
        return carry

    lax.fori_loop(0, NCHUNK, chunk_body, 0)


def kernel(x, embedding, positional_encoding):
    x_flat = x.reshape(TOTAL)
    pos2d = positional_encoding.reshape(SEQ_LEN, MODEL_DIM)
    out = _encode(x_flat, embedding, pos2d)
    return out.reshape(BATCH, SEQ_LEN, MODEL_DIM)


# speedup vs baseline: 1.3887x; 1.3887x over previous
import functools

import jax
import jax.numpy as jnp
from jax import lax
from jax.experimental import pallas as pl
from jax.experimental.pallas import tpu as pltpu
from jax.experimental.pallas import tpu_sc as plsc

MODEL_DIM = 64
SEQ_LEN = 200
BATCH = 4096
LANES = 16
D_VECS = MODEL_DIM // LANES
SCALE = float(MODEL_DIM) ** 0.5

_info = plsc.get_sparse_core_info()
NC, NS = _info.num_cores, _info.num_subcores
NW = NC * NS
XROWS_W = BATCH // NW
CHUNK_X = 4
ROWS = CHUNK_X * SEQ_LEN
NCHUNK = XROWS_W // CHUNK_X
NBUF = 2
G_SPLITS = ((0, 128), (128, 72))

_mesh = plsc.VectorSubcoreMesh(core_axis_name="c", subcore_axis_name="s")


@functools.partial(
    pl.kernel,
    out_type=jax.ShapeDtypeStruct((BATCH, SEQ_LEN, MODEL_DIM), jnp.float32),
    mesh=_mesh,
    compiler_params=pltpu.CompilerParams(use_tc_tiling_on_sc=False),
    scratch_types=[
        pltpu.VMEM((NBUF, CHUNK_X, SEQ_LEN), jnp.int32),
        pltpu.VMEM((NBUF, CHUNK_X, SEQ_LEN, MODEL_DIM), jnp.float32),
        pltpu.VMEM((SEQ_LEN, MODEL_DIM), jnp.float32),
        pltpu.SemaphoreType.DMA,
        pltpu.SemaphoreType.DMA,
        pltpu.SemaphoreType.DMA,
        pltpu.SemaphoreType.DMA,
    ],
)
def _sc_encode(x_hbm, table_hbm, pos_hbm, out_hbm, idx_v, rows_v, pos_v,
               g_sem0, g_sem1, o_sem0, o_sem1):
    wid = lax.axis_index("s") * NC + lax.axis_index("c")
    x_base = wid * XROWS_W
    g_sems = (g_sem0, g_sem1)
    o_sems = (o_sem0, o_sem1)

    pltpu.sync_copy(pos_hbm.at[0], pos_v)

    gathers = [None] * NBUF
    out_cps = [None] * NBUF

    def start_chunk(c):
        k = c % NBUF
        b = x_base + c * CHUNK_X
        pltpu.sync_copy(x_hbm.at[pl.ds(b, CHUNK_X), :], idx_v.at[k])
        cps = []
        for r in range(CHUNK_X):
            for (off, ln) in G_SPLITS:
                cps.append(pltpu.async_copy(
                    table_hbm.at[idx_v.at[k, r, pl.ds(off, ln)]],
                    rows_v.at[k, r, pl.ds(off, ln)],
                    g_sems[k],
                ))
        gathers[k] = cps

    def finish_chunk(c):
        k = c % NBUF
        for cp in gathers[k]:
            cp.wait()

        def body(s, carry):
            for d in range(D_VECS):
                pv = pos_v[s, pl.ds(d * LANES, LANES)]
                for r in range(CHUNK_X):
                    v = rows_v[k, r, s, pl.ds(d * LANES, LANES)]
                    rows_v[k, r, s, pl.ds(d * LANES, LANES)] = v * SCALE + pv
            return carry

        lax.fori_loop(0, SEQ_LEN, body, 0)
        b = x_base + c * CHUNK_X
        out_cps[k] = pltpu.async_copy(
            rows_v.at[k], out_hbm.at[pl.ds(b, CHUNK_X)], o_sems[k])

    for c in range(NCHUNK):
        k = c % NBUF
        if out_cps[k] is not None:
            out_cps[k].wait()
            out_cps[k] = None
        start_chunk(c)
        if c >= 1:
            finish_chunk(c - 1)
    finish_chunk(NCHUNK - 1)
    for k in range(NBUF):
        if out_cps[k] is not None:
            out_cps[k].wait()


def kernel(x, embedding, positional_encoding):
    return _sc_encode(x, embedding, positional_encoding)
